# speedup vs baseline: 1.4410x; 1.4410x over previous
import jax
import jax.numpy as jnp
from jax.experimental import pallas as pl
from jax.experimental.pallas import tpu as pltpu

SEQ_BLOCK = 512


def _add_body(we_ref, pe_ref, out_ref):
    out_ref[...] = we_ref[...] + pe_ref[...][None, :, :]


def kernel(word_embeddings, pe):
    batch, seq_len, d_model = word_embeddings.shape
    n_seq = seq_len // SEQ_BLOCK
    return pl.pallas_call(
        _add_body,
        grid=(n_seq, batch),
        in_specs=[
            pl.BlockSpec((1, SEQ_BLOCK, d_model), lambda s, b: (b, s, 0)),
            pl.BlockSpec((SEQ_BLOCK, d_model), lambda s, b: (s, 0)),
        ],
        out_specs=pl.BlockSpec((1, SEQ_BLOCK, d_model), lambda s, b: (b, s, 0)),
        out_shape=jax.ShapeDtypeStruct((batch, seq_len, d_model), jnp.float32),
        compiler_params=pltpu.CompilerParams(
            dimension_semantics=("arbitrary", "arbitrary"),
        ),
    )(word_embeddings, pe)


# speedup vs baseline: 1.6884x; 1.1717x over previous
import jax
import jax.numpy as jnp
from jax.experimental import pallas as pl
from jax.experimental.pallas import tpu as pltpu

SEQ_BLOCK = 1024


def _add_body(we_ref, pe_ref, out_ref):
    out_ref[...] = we_ref[...] + pe_ref[...][None, :, :]


def kernel(word_embeddings, pe):
    batch, seq_len, d_model = word_embeddings.shape
    n_seq = seq_len // SEQ_BLOCK
    return pl.pallas_call(
        _add_body,
        grid=(n_seq, batch),
        in_specs=[
            pl.BlockSpec((1, SEQ_BLOCK, d_model), lambda s, b: (b, s, 0)),
            pl.BlockSpec((SEQ_BLOCK, d_model), lambda s, b: (s, 0)),
        ],
        out_specs=pl.BlockSpec((1, SEQ_BLOCK, d_model), lambda s, b: (b, s, 0)),
        out_shape=jax.ShapeDtypeStruct((batch, seq_len, d_model), jnp.float32),
        compiler_params=pltpu.CompilerParams(
            dimension_semantics=("arbitrary", "arbitrary"),
        ),
    )(word_embeddings, pe)


# speedup vs baseline: 1.7955x; 1.0634x over previous
import jax
import jax.numpy as jnp
from jax.experimental import pallas as pl
from jax.experimental.pallas import tpu as pltpu

SEQ_BLOCK = 2048


def _add_body(we_ref, pe_ref, out_ref):
    out_ref[...] = we_ref[...] + pe_ref[...][None, :, :]


def kernel(word_embeddings, pe):
    batch, seq_len, d_model = word_embeddings.shape
    n_seq = seq_len // SEQ_BLOCK
    return pl.pallas_call(
        _add_body,
        grid=(n_seq, batch),
        in_specs=[
            pl.BlockSpec((1, SEQ_BLOCK, d_model), lambda s, b: (b, s, 0)),
            pl.BlockSpec((SEQ_BLOCK, d_model), lambda s, b: (s, 0)),
        ],
        out_specs=pl.BlockSpec((1, SEQ_BLOCK, d_model), lambda s, b: (b, s, 0)),
        out_shape=jax.ShapeDtypeStruct((batch, seq_len, d_model), jnp.float32),
        compiler_params=pltpu.CompilerParams(
            dimension_semantics=("arbitrary", "arbitrary"),
        ),
    )(word_embeddings, pe)
